# Initial kernel scaffold; baseline (speedup 1.0000x reference)
import jax
import jax.numpy as jnp
from jax.experimental import pallas as pl


def kernel(x, pos, batch, cluster_weights, R, t, edge_index):
    raise NotImplementedError("write your pallas kernel here")


# speedup vs baseline: 1.4519x; 1.4519x over previous
import functools

import jax
import jax.numpy as jnp
from jax import lax
from jax.experimental import pallas as pl
from jax.experimental.pallas import tpu as pltpu
from jax.experimental.pallas import tpu_sc as plsc

NC = 2
NS = 16
L = 16
NW = NC * NS


def _sc_transf_prop(n_points, chunk):
    steps = n_points // (NW * chunk)
    per_worker = n_points // NW
    mesh = plsc.VectorSubcoreMesh(
        core_axis_name="c", subcore_axis_name="s", num_cores=NC,
        num_subcores=NS)

    @functools.partial(
        pl.kernel,
        mesh=mesh,
        out_type=(
            jax.ShapeDtypeStruct((n_points, 9), jnp.float32),
            jax.ShapeDtypeStruct((n_points, 3), jnp.float32),
            jax.ShapeDtypeStruct((n_points, 3), jnp.float32),
        ),
        scratch_types=[
            pltpu.VMEM((chunk,), jnp.int32),
            pltpu.VMEM((chunk, 16), jnp.float32),
            pltpu.VMEM((chunk, 3), jnp.float32),
            pltpu.VMEM((chunk, 3), jnp.float32),
            pltpu.VMEM((chunk, 9), jnp.float32),
            pltpu.VMEM((chunk, 3), jnp.float32),
            pltpu.VMEM((chunk, 3), jnp.float32),
            pltpu.SemaphoreType.DMA,
        ],
        compiler_params=pltpu.CompilerParams(
            needs_layout_passes=False, use_tc_tiling_on_sc=False),
    )
    def sc_kernel(tbl_hbm, idx_hbm, pos_hbm, x_hbm,
                  rp_hbm, tp_hbm, xo_hbm,
                  idx_v, tbl_v, pos_v, x_v, rp_v, tp_v, xo_v, sem):
        wid = lax.axis_index("s") * NC + lax.axis_index("c")
        lane = lax.iota(jnp.int32, L)

        def step(s, _):
            base = wid * per_worker + s * chunk
            pltpu.sync_copy(idx_hbm.at[pl.ds(base, chunk)], idx_v)
            pltpu.sync_copy(pos_hbm.at[pl.ds(base, chunk)], pos_v)
            pltpu.sync_copy(x_hbm.at[pl.ds(base, chunk)], x_v)
            pltpu.async_copy(tbl_hbm.at[idx_v], tbl_v, sem).wait()

            def group(g, _):
                rows = lane + g * L

                def col(j):
                    return jnp.full((L,), j, jnp.int32)

                def gat(ref, j):
                    return plsc.load_gather(ref, [rows, col(j)])

                r = [gat(tbl_v, j) for j in range(9)]
                t0, t1, t2 = (gat(tbl_v, j) for j in range(9, 12))
                w = gat(tbl_v, 12)
                p0, p1, p2 = (gat(pos_v, j) for j in range(3))
                x0, x1, x2 = (gat(x_v, j) for j in range(3))

                xp0 = r[0] * p0 + r[1] * p1 + r[2] * p2 + t0
                xp1 = r[3] * p0 + r[4] * p1 + r[5] * p2 + t1
                xp2 = r[6] * p0 + r[7] * p1 + r[8] * p2 + t2
                omw = 1.0 - w
                xo0 = x0 * w + xp0 * omw
                xo1 = x1 * w + xp1 * omw
                xo2 = x2 * w + xp2 * omw

                for j in range(9):
                    plsc.store_scatter(rp_v, [rows, col(j)], r[j])
                for j, v in enumerate((t0, t1, t2)):
                    plsc.store_scatter(tp_v, [rows, col(j)], v)
                for j, v in enumerate((xo0, xo1, xo2)):
                    plsc.store_scatter(xo_v, [rows, col(j)], v)

            lax.fori_loop(0, chunk // L, group, None)

            pltpu.sync_copy(rp_v, rp_hbm.at[pl.ds(base, chunk)])
            pltpu.sync_copy(tp_v, tp_hbm.at[pl.ds(base, chunk)])
            pltpu.sync_copy(xo_v, xo_hbm.at[pl.ds(base, chunk)])

        lax.fori_loop(0, steps, step, None)

    return sc_kernel


def kernel(x, pos, batch, cluster_weights, R, t, edge_index):
    n = x.shape[0]
    m = R.shape[0]
    cluster_idx = edge_index[1].astype(jnp.int32)
    table = jnp.concatenate(
        [R.reshape(m, 9), t, cluster_weights,
         jnp.zeros((m, 3), jnp.float32)], axis=1)
    rp, tp, xo = _sc_transf_prop(n, 2000)(table, cluster_idx, pos, x)
    return rp.reshape(n, 3, 3), tp, xo


# speedup vs baseline: 5.4162x; 3.7305x over previous
import functools

import jax
import jax.numpy as jnp
from jax import lax
from jax.experimental import pallas as pl
from jax.experimental.pallas import tpu as pltpu
from jax.experimental.pallas import tpu_sc as plsc

NC = 2
NS = 16
L = 16
NW = NC * NS


def _sc_transf_prop(n_points, chunk):
    steps = n_points // (NW * chunk)
    per_worker = n_points // NW
    mesh = plsc.VectorSubcoreMesh(
        core_axis_name="c", subcore_axis_name="s", num_cores=NC,
        num_subcores=NS)

    @functools.partial(
        pl.kernel,
        mesh=mesh,
        out_type=(
            jax.ShapeDtypeStruct((9, n_points), jnp.float32),
            jax.ShapeDtypeStruct((3, n_points), jnp.float32),
            jax.ShapeDtypeStruct((3, n_points), jnp.float32),
        ),
        scratch_types=[
            pltpu.VMEM((chunk,), jnp.int32),
            pltpu.VMEM((chunk, 16), jnp.float32),
            pltpu.VMEM((3, chunk), jnp.float32),
            pltpu.VMEM((3, chunk), jnp.float32),
            pltpu.VMEM((9, chunk), jnp.float32),
            pltpu.VMEM((3, chunk), jnp.float32),
            pltpu.VMEM((3, chunk), jnp.float32),
            pltpu.SemaphoreType.DMA,
        ],
        compiler_params=pltpu.CompilerParams(
            needs_layout_passes=False, use_tc_tiling_on_sc=False),
    )
    def sc_kernel(tbl_hbm, idx_hbm, pos_hbm, x_hbm,
                  rp_hbm, tp_hbm, xo_hbm,
                  idx_v, tbl_v, pos_v, x_v, rp_v, tp_v, xo_v, sem):
        wid = lax.axis_index("s") * NC + lax.axis_index("c")
        lane = lax.iota(jnp.int32, L)

        def step(s, _):
            base = wid * per_worker + s * chunk
            pltpu.sync_copy(idx_hbm.at[pl.ds(base, chunk)], idx_v)
            pltpu.sync_copy(pos_hbm.at[:, pl.ds(base, chunk)], pos_v)
            pltpu.sync_copy(x_hbm.at[:, pl.ds(base, chunk)], x_v)
            pltpu.async_copy(tbl_hbm.at[idx_v], tbl_v, sem).wait()

            def group(g, _):
                o = g * L
                rows = lane + o

                def gat(j):
                    return plsc.load_gather(
                        tbl_v, [rows, jnp.full((L,), j, jnp.int32)])

                r = [gat(j) for j in range(9)]
                t0, t1, t2 = (gat(j) for j in range(9, 12))
                w = gat(12)
                p0 = pos_v[0, pl.ds(o, L)]
                p1 = pos_v[1, pl.ds(o, L)]
                p2 = pos_v[2, pl.ds(o, L)]
                x0 = x_v[0, pl.ds(o, L)]
                x1 = x_v[1, pl.ds(o, L)]
                x2 = x_v[2, pl.ds(o, L)]

                xp0 = r[0] * p0 + r[1] * p1 + r[2] * p2 + t0
                xp1 = r[3] * p0 + r[4] * p1 + r[5] * p2 + t1
                xp2 = r[6] * p0 + r[7] * p1 + r[8] * p2 + t2
                omw = 1.0 - w
                for j in range(9):
                    rp_v[j, pl.ds(o, L)] = r[j]
                tp_v[0, pl.ds(o, L)] = t0
                tp_v[1, pl.ds(o, L)] = t1
                tp_v[2, pl.ds(o, L)] = t2
                xo_v[0, pl.ds(o, L)] = x0 * w + xp0 * omw
                xo_v[1, pl.ds(o, L)] = x1 * w + xp1 * omw
                xo_v[2, pl.ds(o, L)] = x2 * w + xp2 * omw

            lax.fori_loop(0, chunk // L, group, None)

            pltpu.sync_copy(rp_v, rp_hbm.at[:, pl.ds(base, chunk)])
            pltpu.sync_copy(tp_v, tp_hbm.at[:, pl.ds(base, chunk)])
            pltpu.sync_copy(xo_v, xo_hbm.at[:, pl.ds(base, chunk)])

        lax.fori_loop(0, steps, step, None)

    return sc_kernel


def kernel(x, pos, batch, cluster_weights, R, t, edge_index):
    n = x.shape[0]
    m = R.shape[0]
    cluster_idx = edge_index[1].astype(jnp.int32)
    table = jnp.concatenate(
        [R.reshape(m, 9), t, cluster_weights,
         jnp.zeros((m, 3), jnp.float32)], axis=1)
    rp, tp, xo = _sc_transf_prop(n, 2000)(
        table, cluster_idx, pos.T, x.T)
    return (rp.reshape(3, 3, n).transpose(2, 0, 1), tp.T, xo.T)


# speedup vs baseline: 14.6676x; 2.7081x over previous
import functools

import jax
import jax.numpy as jnp
from jax import lax
from jax.experimental import pallas as pl
from jax.experimental.pallas import tpu as pltpu
from jax.experimental.pallas import tpu_sc as plsc

NC = 2
NS = 16
L = 16
NW = NC * NS
BLK = 128
CHUNK = 1280
NBLK = CHUNK // BLK


def _sc_transf_prop(n_points):
    n_blocks = n_points // BLK
    n_chunks = n_points // CHUNK
    steps = -(-n_chunks // NW)
    mesh = plsc.VectorSubcoreMesh(
        core_axis_name="c", subcore_axis_name="s", num_cores=NC,
        num_subcores=NS)

    @functools.partial(
        pl.kernel,
        mesh=mesh,
        out_type=(
            jax.ShapeDtypeStruct((3, n_blocks, 4, BLK), jnp.float32),
            jax.ShapeDtypeStruct((n_blocks, 4, BLK), jnp.float32),
            jax.ShapeDtypeStruct((n_blocks, 4, BLK), jnp.float32),
        ),
        scratch_types=[
            pltpu.VMEM((CHUNK,), jnp.int32),
            pltpu.VMEM((CHUNK, 16), jnp.float32),
            pltpu.VMEM((3, CHUNK), jnp.float32),
            pltpu.VMEM((3, CHUNK), jnp.float32),
            pltpu.VMEM((3, 3, NBLK, BLK), jnp.float32),
            pltpu.VMEM((3, NBLK, BLK), jnp.float32),
            pltpu.VMEM((3, NBLK, BLK), jnp.float32),
            pltpu.SemaphoreType.DMA,
        ],
        compiler_params=pltpu.CompilerParams(
            needs_layout_passes=False, use_tc_tiling_on_sc=False),
    )
    def sc_kernel(tbl_hbm, idx_hbm, pos_hbm, x_hbm,
                  rp_hbm, tp_hbm, xo_hbm,
                  idx_v, tbl_v, pos_v, x_v, rp_v, tp_v, xo_v, sem):
        wid = lax.axis_index("s") * NC + lax.axis_index("c")
        lane = lax.iota(jnp.int32, L)

        def step(k, _):
            cid = k * NW + wid

            @pl.when(cid < n_chunks)
            def _():
                base = cid * CHUNK
                b0 = cid * NBLK
                pltpu.sync_copy(idx_hbm.at[pl.ds(base, CHUNK)], idx_v)
                pltpu.sync_copy(pos_hbm.at[:, pl.ds(base, CHUNK)], pos_v)
                pltpu.sync_copy(x_hbm.at[:, pl.ds(base, CHUNK)], x_v)
                pltpu.async_copy(tbl_hbm.at[idx_v], tbl_v, sem).wait()

                def group(g, _):
                    o = g * L
                    blk = g // (BLK // L)
                    l0 = (g % (BLK // L)) * L
                    rows = lane + o

                    def gat(j):
                        return plsc.load_gather(
                            tbl_v, [rows, jnp.full((L,), j, jnp.int32)])

                    r = [gat(j) for j in range(9)]
                    t0, t1, t2 = (gat(j) for j in range(9, 12))
                    w = gat(12)
                    p0 = pos_v[0, pl.ds(o, L)]
                    p1 = pos_v[1, pl.ds(o, L)]
                    p2 = pos_v[2, pl.ds(o, L)]
                    x0 = x_v[0, pl.ds(o, L)]
                    x1 = x_v[1, pl.ds(o, L)]
                    x2 = x_v[2, pl.ds(o, L)]

                    xp0 = r[0] * p0 + r[1] * p1 + r[2] * p2 + t0
                    xp1 = r[3] * p0 + r[4] * p1 + r[5] * p2 + t1
                    xp2 = r[6] * p0 + r[7] * p1 + r[8] * p2 + t2
                    omw = 1.0 - w
                    for rr in range(3):
                        for cc in range(3):
                            rp_v[rr, cc, blk, pl.ds(l0, L)] = r[3 * rr + cc]
                    tp_v[0, blk, pl.ds(l0, L)] = t0
                    tp_v[1, blk, pl.ds(l0, L)] = t1
                    tp_v[2, blk, pl.ds(l0, L)] = t2
                    xo_v[0, blk, pl.ds(l0, L)] = x0 * w + xp0 * omw
                    xo_v[1, blk, pl.ds(l0, L)] = x1 * w + xp1 * omw
                    xo_v[2, blk, pl.ds(l0, L)] = x2 * w + xp2 * omw

                lax.fori_loop(0, CHUNK // L, group, None)

                for rr in range(3):
                    for cc in range(3):
                        pltpu.sync_copy(
                            rp_v.at[rr, cc],
                            rp_hbm.at[rr, pl.ds(b0, NBLK), cc, :])
                for cc in range(3):
                    pltpu.sync_copy(
                        tp_v.at[cc], tp_hbm.at[pl.ds(b0, NBLK), cc, :])
                    pltpu.sync_copy(
                        xo_v.at[cc], xo_hbm.at[pl.ds(b0, NBLK), cc, :])

        lax.fori_loop(0, steps, step, None)

    return sc_kernel


def kernel(x, pos, batch, cluster_weights, R, t, edge_index):
    n = x.shape[0]
    m = R.shape[0]
    nb = n // BLK
    cluster_idx = edge_index[1].astype(jnp.int32)
    table = jnp.concatenate(
        [R.reshape(m, 9), t, cluster_weights,
         jnp.zeros((m, 3), jnp.float32)], axis=1)
    rp4, tp4, xo4 = _sc_transf_prop(n)(table, cluster_idx, pos.T, x.T)
    r_point = rp4[:, :, :3, :].transpose(1, 3, 0, 2).reshape(n, 3, 3)
    t_point = tp4[:, :3, :].transpose(0, 2, 1).reshape(n, 3)
    x_out = xo4[:, :3, :].transpose(0, 2, 1).reshape(n, 3)
    return (r_point, t_point, x_out)


# speedup vs baseline: 34.9842x; 2.3851x over previous
import functools

import jax
import jax.numpy as jnp
from jax import lax
from jax.experimental import pallas as pl
from jax.experimental.pallas import tpu as pltpu
from jax.experimental.pallas import tpu_sc as plsc

NC = 2
NS = 16
L = 16
NW = NC * NS
BLK = 128
CHUNK = 1280
NBLK = CHUNK // BLK


def _sc_transf_prop(n_points):
    n_blocks = n_points // BLK
    n_chunks = n_points // CHUNK
    steps = -(-n_chunks // NW)
    mesh = plsc.VectorSubcoreMesh(
        core_axis_name="c", subcore_axis_name="s", num_cores=NC,
        num_subcores=NS)

    @functools.partial(
        pl.kernel,
        mesh=mesh,
        out_type=(
            jax.ShapeDtypeStruct((3, n_blocks, 4, BLK), jnp.float32),
            jax.ShapeDtypeStruct((n_blocks, 4, BLK), jnp.float32),
            jax.ShapeDtypeStruct((n_blocks, 4, BLK), jnp.float32),
        ),
        scratch_types=[
            pltpu.VMEM((CHUNK,), jnp.int32),
            pltpu.VMEM((CHUNK, 16), jnp.float32),
            pltpu.VMEM((NBLK, 3, BLK), jnp.float32),
            pltpu.VMEM((NBLK, 3, BLK), jnp.float32),
            pltpu.VMEM((3, 3, NBLK, BLK), jnp.float32),
            pltpu.VMEM((3, NBLK, BLK), jnp.float32),
            pltpu.VMEM((3, NBLK, BLK), jnp.float32),
            pltpu.SemaphoreType.DMA,
            pltpu.SemaphoreType.DMA,
        ],
        compiler_params=pltpu.CompilerParams(
            needs_layout_passes=False, use_tc_tiling_on_sc=False),
    )
    def sc_kernel(tbl_hbm, idx_hbm, pos_hbm, x_hbm,
                  rp_hbm, tp_hbm, xo_hbm,
                  idx_v, tbl_v, pos_v, x_v, rp_v, tp_v, xo_v,
                  sem_in, sem_out):
        wid = lax.axis_index("s") * NC + lax.axis_index("c")
        lane = lax.iota(jnp.int32, L)

        def step(k, _):
            cid = k * NW + wid

            @pl.when(cid < n_chunks)
            def _():
                base = cid * CHUNK
                b0 = cid * NBLK
                pltpu.sync_copy(idx_hbm.at[pl.ds(base, CHUNK)], idx_v)
                pends = [
                    pltpu.async_copy(tbl_hbm.at[idx_v], tbl_v, sem_in),
                    pltpu.async_copy(
                        pos_hbm.at[pl.ds(b0, NBLK)], pos_v, sem_in),
                    pltpu.async_copy(
                        x_hbm.at[pl.ds(b0, NBLK)], x_v, sem_in),
                ]
                for p in pends:
                    p.wait()

                def group(g, _):
                    blk = g // (BLK // L)
                    l0 = (g % (BLK // L)) * L
                    rows = lane + g * L

                    def gat(j):
                        return plsc.load_gather(
                            tbl_v, [rows, jnp.full((L,), j, jnp.int32)])

                    r = [gat(j) for j in range(9)]
                    t0, t1, t2 = (gat(j) for j in range(9, 12))
                    w = gat(12)
                    p0 = pos_v[blk, 0, pl.ds(l0, L)]
                    p1 = pos_v[blk, 1, pl.ds(l0, L)]
                    p2 = pos_v[blk, 2, pl.ds(l0, L)]
                    x0 = x_v[blk, 0, pl.ds(l0, L)]
                    x1 = x_v[blk, 1, pl.ds(l0, L)]
                    x2 = x_v[blk, 2, pl.ds(l0, L)]

                    xp0 = r[0] * p0 + r[1] * p1 + r[2] * p2 + t0
                    xp1 = r[3] * p0 + r[4] * p1 + r[5] * p2 + t1
                    xp2 = r[6] * p0 + r[7] * p1 + r[8] * p2 + t2
                    omw = 1.0 - w
                    for rr in range(3):
                        for cc in range(3):
                            rp_v[rr, cc, blk, pl.ds(l0, L)] = r[3 * rr + cc]
                    tp_v[0, blk, pl.ds(l0, L)] = t0
                    tp_v[1, blk, pl.ds(l0, L)] = t1
                    tp_v[2, blk, pl.ds(l0, L)] = t2
                    xo_v[0, blk, pl.ds(l0, L)] = x0 * w + xp0 * omw
                    xo_v[1, blk, pl.ds(l0, L)] = x1 * w + xp1 * omw
                    xo_v[2, blk, pl.ds(l0, L)] = x2 * w + xp2 * omw

                lax.fori_loop(0, CHUNK // L, group, None)

                pends = []
                for rr in range(3):
                    for cc in range(3):
                        pends.append(pltpu.async_copy(
                            rp_v.at[rr, cc],
                            rp_hbm.at[rr, pl.ds(b0, NBLK), cc, :],
                            sem_out))
                for cc in range(3):
                    pends.append(pltpu.async_copy(
                        tp_v.at[cc], tp_hbm.at[pl.ds(b0, NBLK), cc, :],
                        sem_out))
                    pends.append(pltpu.async_copy(
                        xo_v.at[cc], xo_hbm.at[pl.ds(b0, NBLK), cc, :],
                        sem_out))
                for p in pends:
                    p.wait()

        lax.fori_loop(0, steps, step, None)

    return sc_kernel


def kernel(x, pos, batch, cluster_weights, R, t, edge_index):
    n = x.shape[0]
    m = R.shape[0]
    nb = n // BLK
    cluster_idx = edge_index[1].astype(jnp.int32)
    table = jnp.concatenate(
        [R.reshape(m, 9), t, cluster_weights,
         jnp.zeros((m, 3), jnp.float32)], axis=1)
    pos3 = pos.reshape(nb, BLK, 3).transpose(0, 2, 1)
    x3 = x.reshape(nb, BLK, 3).transpose(0, 2, 1)
    rp4, tp4, xo4 = _sc_transf_prop(n)(table, cluster_idx, pos3, x3)
    r_point = rp4[:, :, :3, :].transpose(1, 3, 0, 2).reshape(n, 3, 3)
    t_point = tp4[:, :3, :].transpose(0, 2, 1).reshape(n, 3)
    x_out = xo4[:, :3, :].transpose(0, 2, 1).reshape(n, 3)
    return (r_point, t_point, x_out)


# speedup vs baseline: 43.2851x; 1.2373x over previous
import functools

import jax
import jax.numpy as jnp
from jax import lax
from jax.experimental import pallas as pl
from jax.experimental.pallas import tpu as pltpu
from jax.experimental.pallas import tpu_sc as plsc

NC = 2
NS = 16
L = 16
NW = NC * NS
BLK = 128
CHUNK = 1280
NBLK = CHUNK // BLK


def _sc_transf_prop(n_points):
    n_blocks = n_points // BLK
    n_chunks = n_points // CHUNK
    steps = -(-n_chunks // NW)
    steps += steps % 2
    mesh = plsc.VectorSubcoreMesh(
        core_axis_name="c", subcore_axis_name="s", num_cores=NC,
        num_subcores=NS)

    buf_set = [
        pltpu.VMEM((NBLK, BLK), jnp.int32),
        pltpu.VMEM((CHUNK,), jnp.int32),
        pltpu.VMEM((CHUNK, 16), jnp.float32),
        pltpu.VMEM((NBLK, 3, BLK), jnp.float32),
        pltpu.VMEM((NBLK, 3, BLK), jnp.float32),
        pltpu.VMEM((3, 3, NBLK, BLK), jnp.float32),
        pltpu.VMEM((3, NBLK, BLK), jnp.float32),
        pltpu.VMEM((3, NBLK, BLK), jnp.float32),
        pltpu.SemaphoreType.DMA,
        pltpu.SemaphoreType.DMA,
        pltpu.SemaphoreType.DMA,
    ]
    NB = len(buf_set)

    @functools.partial(
        pl.kernel,
        mesh=mesh,
        out_type=(
            jax.ShapeDtypeStruct((3, n_blocks, 4, BLK), jnp.float32),
            jax.ShapeDtypeStruct((n_blocks, 4, BLK), jnp.float32),
            jax.ShapeDtypeStruct((n_blocks, 4, BLK), jnp.float32),
        ),
        scratch_types=buf_set + buf_set,
        compiler_params=pltpu.CompilerParams(
            needs_layout_passes=False, use_tc_tiling_on_sc=False),
    )
    def sc_kernel(tbl_hbm, edge_hbm, pos_hbm, x_hbm,
                  rp_hbm, tp_hbm, xo_hbm, *bufs):
        B = [bufs[:NB], bufs[NB:]]
        wid = lax.axis_index("s") * NC + lax.axis_index("c")
        lane = lax.iota(jnp.int32, L)

        def in_triples(cid, b):
            b0 = cid * NBLK
            sem = B[b][8]
            return [
                (edge_hbm.at[pl.ds(b0, NBLK), 1, :], B[b][0], sem),
                (pos_hbm.at[pl.ds(b0, NBLK)], B[b][3], sem),
                (x_hbm.at[pl.ds(b0, NBLK)], B[b][4], sem),
            ]

        def repack_idx(b):
            i2, i1 = B[b][0], B[b][1]

            def rep(g, _):
                blk = g // (BLK // L)
                l0 = (g % (BLK // L)) * L
                i1[pl.ds(g * L, L)] = i2[blk, pl.ds(l0, L)]

            lax.fori_loop(0, CHUNK // L, rep, None)

        def gather_args(b):
            return (tbl_hbm.at[B[b][1]], B[b][2], B[b][9])

        def out_triples(cid, b):
            b0 = cid * NBLK
            rp_v, tp_v, xo_v, sem = B[b][5], B[b][6], B[b][7], B[b][10]
            trips = []
            for rr in range(3):
                for cc in range(3):
                    trips.append((rp_v.at[rr, cc],
                                  rp_hbm.at[rr, pl.ds(b0, NBLK), cc, :],
                                  sem))
            for cc in range(3):
                trips.append((tp_v.at[cc],
                              tp_hbm.at[pl.ds(b0, NBLK), cc, :], sem))
                trips.append((xo_v.at[cc],
                              xo_hbm.at[pl.ds(b0, NBLK), cc, :], sem))
            return trips

        def compute(b):
            tbl_v, pos_v, x_v = B[b][2], B[b][3], B[b][4]
            rp_v, tp_v, xo_v = B[b][5], B[b][6], B[b][7]

            def group(g, _):
                blk = g // (BLK // L)
                l0 = (g % (BLK // L)) * L
                rows = lane + g * L

                def gat(j):
                    return plsc.load_gather(
                        tbl_v, [rows, jnp.full((L,), j, jnp.int32)])

                r = [gat(j) for j in range(9)]
                t0, t1, t2 = (gat(j) for j in range(9, 12))
                w = gat(12)
                p0 = pos_v[blk, 0, pl.ds(l0, L)]
                p1 = pos_v[blk, 1, pl.ds(l0, L)]
                p2 = pos_v[blk, 2, pl.ds(l0, L)]
                x0 = x_v[blk, 0, pl.ds(l0, L)]
                x1 = x_v[blk, 1, pl.ds(l0, L)]
                x2 = x_v[blk, 2, pl.ds(l0, L)]

                xp0 = r[0] * p0 + r[1] * p1 + r[2] * p2 + t0
                xp1 = r[3] * p0 + r[4] * p1 + r[5] * p2 + t1
                xp2 = r[6] * p0 + r[7] * p1 + r[8] * p2 + t2
                omw = 1.0 - w
                for rr in range(3):
                    for cc in range(3):
                        rp_v[rr, cc, blk, pl.ds(l0, L)] = r[3 * rr + cc]
                tp_v[0, blk, pl.ds(l0, L)] = t0
                tp_v[1, blk, pl.ds(l0, L)] = t1
                tp_v[2, blk, pl.ds(l0, L)] = t2
                xo_v[0, blk, pl.ds(l0, L)] = x0 * w + xp0 * omw
                xo_v[1, blk, pl.ds(l0, L)] = x1 * w + xp1 * omw
                xo_v[2, blk, pl.ds(l0, L)] = x2 * w + xp2 * omw

            lax.fori_loop(0, CHUNK // L, group, None)

        for s, d, sm in in_triples(wid, 0):
            pltpu.async_copy(s, d, sm)
        s, d, sm = in_triples(wid, 0)[0]
        pltpu.make_async_copy(s, d, sm).wait()
        repack_idx(0)
        pltpu.async_copy(*gather_args(0))

        def half_step(k, b):
            cid = k * NW + wid
            nxt = cid + NW
            prv = cid - 2 * NW

            @pl.when(nxt < n_chunks)
            def _issue_next_in():
                for s, d, sm in in_triples(nxt, 1 - b):
                    pltpu.async_copy(s, d, sm)

            @pl.when((k >= 2) & (prv >= 0) & (prv < n_chunks))
            def _drain_prev_out():
                for s, d, sm in out_triples(prv, b):
                    pltpu.make_async_copy(s, d, sm).wait()

            @pl.when(cid < n_chunks)
            def _compute():
                g_s, g_d, g_sm = gather_args(b)
                pltpu.make_async_copy(g_s, g_d, g_sm).wait()
                for s, d, sm in in_triples(cid, b)[1:]:
                    pltpu.make_async_copy(s, d, sm).wait()
                compute(b)
                for s, d, sm in out_triples(cid, b):
                    pltpu.async_copy(s, d, sm)

            @pl.when(nxt < n_chunks)
            def _issue_next_gather():
                s, d, sm = in_triples(nxt, 1 - b)[0]
                pltpu.make_async_copy(s, d, sm).wait()
                repack_idx(1 - b)
                pltpu.async_copy(*gather_args(1 - b))

        def pair(j, _):
            half_step(2 * j, 0)
            half_step(2 * j + 1, 1)

        lax.fori_loop(0, steps // 2, pair, None)

        for b in range(2):
            kd = steps - 2 + b
            cid = kd * NW + wid

            @pl.when(cid < n_chunks)
            def _drain_tail(cid=cid, b=b):
                for s, d, sm in out_triples(cid, b):
                    pltpu.make_async_copy(s, d, sm).wait()

    return sc_kernel


def kernel(x, pos, batch, cluster_weights, R, t, edge_index):
    n = x.shape[0]
    m = R.shape[0]
    nb = n // BLK
    edge3 = (edge_index.astype(jnp.int32).T
             .reshape(nb, BLK, 2).transpose(0, 2, 1))
    table = jnp.concatenate(
        [R.reshape(m, 9), t, cluster_weights,
         jnp.zeros((m, 3), jnp.float32)], axis=1)
    pos3 = pos.reshape(nb, BLK, 3).transpose(0, 2, 1)
    x3 = x.reshape(nb, BLK, 3).transpose(0, 2, 1)
    rp4, tp4, xo4 = _sc_transf_prop(n)(table, edge3, pos3, x3)
    r_point = rp4[:, :, :3, :].transpose(1, 3, 0, 2).reshape(n, 3, 3)
    t_point = tp4[:, :3, :].transpose(0, 2, 1).reshape(n, 3)
    x_out = xo4[:, :3, :].transpose(0, 2, 1).reshape(n, 3)
    return (r_point, t_point, x_out)


# speedup vs baseline: 44.7977x; 1.0349x over previous
import functools

import jax
import jax.numpy as jnp
from jax import lax
from jax.experimental import pallas as pl
from jax.experimental.pallas import tpu as pltpu
from jax.experimental.pallas import tpu_sc as plsc

NC = 2
NS = 16
L = 16
NW = NC * NS
BLK = 128
CHUNK = 1280
NBLK = CHUNK // BLK


def _sc_transf_prop(n_points):
    n_blocks = n_points // BLK
    n_chunks = n_points // CHUNK
    steps = -(-n_chunks // NW)
    steps += steps % 2
    mesh = plsc.VectorSubcoreMesh(
        core_axis_name="c", subcore_axis_name="s", num_cores=NC,
        num_subcores=NS)

    buf_set = [
        pltpu.VMEM((NBLK, BLK), jnp.int32),
        pltpu.VMEM((CHUNK,), jnp.int32),
        pltpu.VMEM((CHUNK, 16), jnp.float32),
        pltpu.VMEM((NBLK, 3, BLK), jnp.float32),
        pltpu.VMEM((NBLK, 3, BLK), jnp.float32),
        pltpu.VMEM((3, NBLK, 4, BLK), jnp.float32),
        pltpu.VMEM((NBLK, 4, BLK), jnp.float32),
        pltpu.VMEM((NBLK, 4, BLK), jnp.float32),
        pltpu.SemaphoreType.DMA,
        pltpu.SemaphoreType.DMA,
        pltpu.SemaphoreType.DMA,
    ]
    NB = len(buf_set)

    @functools.partial(
        pl.kernel,
        mesh=mesh,
        out_type=(
            jax.ShapeDtypeStruct((3, n_blocks, 4, BLK), jnp.float32),
            jax.ShapeDtypeStruct((n_blocks, 4, BLK), jnp.float32),
            jax.ShapeDtypeStruct((n_blocks, 4, BLK), jnp.float32),
        ),
        scratch_types=buf_set + buf_set,
        compiler_params=pltpu.CompilerParams(
            needs_layout_passes=False, use_tc_tiling_on_sc=False),
    )
    def sc_kernel(tbl_hbm, edge_hbm, pos_hbm, x_hbm,
                  rp_hbm, tp_hbm, xo_hbm, *bufs):
        B = [bufs[:NB], bufs[NB:]]
        wid = lax.axis_index("s") * NC + lax.axis_index("c")
        lane = lax.iota(jnp.int32, L)

        def in_triples(cid, b):
            b0 = cid * NBLK
            sem = B[b][8]
            return [
                (edge_hbm.at[pl.ds(b0, NBLK), 1, :], B[b][0], sem),
                (pos_hbm.at[pl.ds(b0, NBLK)], B[b][3], sem),
                (x_hbm.at[pl.ds(b0, NBLK)], B[b][4], sem),
            ]

        def repack_idx(b):
            i2, i1 = B[b][0], B[b][1]

            def rep(g, _):
                blk = g // (BLK // L)
                l0 = (g % (BLK // L)) * L
                i1[pl.ds(g * L, L)] = i2[blk, pl.ds(l0, L)]

            lax.fori_loop(0, CHUNK // L, rep, None)

        def gather_args(b):
            return (tbl_hbm.at[B[b][1]], B[b][2], B[b][9])

        def out_triples(cid, b):
            b0 = cid * NBLK
            rp_v, tp_v, xo_v, sem = B[b][5], B[b][6], B[b][7], B[b][10]
            trips = [(rp_v.at[rr], rp_hbm.at[rr, pl.ds(b0, NBLK)], sem)
                     for rr in range(3)]
            trips.append((tp_v, tp_hbm.at[pl.ds(b0, NBLK)], sem))
            trips.append((xo_v, xo_hbm.at[pl.ds(b0, NBLK)], sem))
            return trips

        def compute(b):
            tbl_v, pos_v, x_v = B[b][2], B[b][3], B[b][4]
            rp_v, tp_v, xo_v = B[b][5], B[b][6], B[b][7]

            def group(g, _):
                blk = g // (BLK // L)
                l0 = (g % (BLK // L)) * L
                rows = lane + g * L

                def gat(j):
                    return plsc.load_gather(
                        tbl_v, [rows, jnp.full((L,), j, jnp.int32)])

                r = [gat(j) for j in range(9)]
                t0, t1, t2 = (gat(j) for j in range(9, 12))
                w = gat(12)
                p0 = pos_v[blk, 0, pl.ds(l0, L)]
                p1 = pos_v[blk, 1, pl.ds(l0, L)]
                p2 = pos_v[blk, 2, pl.ds(l0, L)]
                x0 = x_v[blk, 0, pl.ds(l0, L)]
                x1 = x_v[blk, 1, pl.ds(l0, L)]
                x2 = x_v[blk, 2, pl.ds(l0, L)]

                xp0 = r[0] * p0 + r[1] * p1 + r[2] * p2 + t0
                xp1 = r[3] * p0 + r[4] * p1 + r[5] * p2 + t1
                xp2 = r[6] * p0 + r[7] * p1 + r[8] * p2 + t2
                omw = 1.0 - w
                for rr in range(3):
                    for cc in range(3):
                        rp_v[rr, blk, cc, pl.ds(l0, L)] = r[3 * rr + cc]
                tp_v[blk, 0, pl.ds(l0, L)] = t0
                tp_v[blk, 1, pl.ds(l0, L)] = t1
                tp_v[blk, 2, pl.ds(l0, L)] = t2
                xo_v[blk, 0, pl.ds(l0, L)] = x0 * w + xp0 * omw
                xo_v[blk, 1, pl.ds(l0, L)] = x1 * w + xp1 * omw
                xo_v[blk, 2, pl.ds(l0, L)] = x2 * w + xp2 * omw

            plsc.parallel_loop(0, CHUNK // L, unroll=2)(
                lambda g: group(g, None))

        for s, d, sm in in_triples(wid, 0):
            pltpu.async_copy(s, d, sm)
        s, d, sm = in_triples(wid, 0)[0]
        pltpu.make_async_copy(s, d, sm).wait()
        repack_idx(0)
        pltpu.async_copy(*gather_args(0))

        def half_step(k, b):
            cid = k * NW + wid
            nxt = cid + NW
            prv = cid - 2 * NW

            @pl.when(nxt < n_chunks)
            def _issue_next_in():
                for s, d, sm in in_triples(nxt, 1 - b):
                    pltpu.async_copy(s, d, sm)

            @pl.when((k >= 2) & (prv >= 0) & (prv < n_chunks))
            def _drain_prev_out():
                for s, d, sm in out_triples(prv, b):
                    pltpu.make_async_copy(s, d, sm).wait()

            @pl.when(cid < n_chunks)
            def _compute():
                g_s, g_d, g_sm = gather_args(b)
                pltpu.make_async_copy(g_s, g_d, g_sm).wait()
                for s, d, sm in in_triples(cid, b)[1:]:
                    pltpu.make_async_copy(s, d, sm).wait()
                compute(b)
                for s, d, sm in out_triples(cid, b):
                    pltpu.async_copy(s, d, sm)

            @pl.when(nxt < n_chunks)
            def _issue_next_gather():
                s, d, sm = in_triples(nxt, 1 - b)[0]
                pltpu.make_async_copy(s, d, sm).wait()
                repack_idx(1 - b)
                pltpu.async_copy(*gather_args(1 - b))

        def pair(j, _):
            half_step(2 * j, 0)
            half_step(2 * j + 1, 1)

        lax.fori_loop(0, steps // 2, pair, None)

        for b in range(2):
            kd = steps - 2 + b
            cid = kd * NW + wid

            @pl.when(cid < n_chunks)
            def _drain_tail(cid=cid, b=b):
                for s, d, sm in out_triples(cid, b):
                    pltpu.make_async_copy(s, d, sm).wait()

    return sc_kernel


def kernel(x, pos, batch, cluster_weights, R, t, edge_index):
    n = x.shape[0]
    m = R.shape[0]
    nb = n // BLK
    edge3 = (edge_index.astype(jnp.int32).T
             .reshape(nb, BLK, 2).transpose(0, 2, 1))
    table = jnp.concatenate(
        [R.reshape(m, 9), t, cluster_weights,
         jnp.zeros((m, 3), jnp.float32)], axis=1)
    pos3 = pos.reshape(nb, BLK, 3).transpose(0, 2, 1)
    x3 = x.reshape(nb, BLK, 3).transpose(0, 2, 1)
    rp4, tp4, xo4 = _sc_transf_prop(n)(table, edge3, pos3, x3)
    r_point = rp4[:, :, :3, :].transpose(1, 3, 0, 2).reshape(n, 3, 3)
    t_point = tp4[:, :3, :].transpose(0, 2, 1).reshape(n, 3)
    x_out = xo4[:, :3, :].transpose(0, 2, 1).reshape(n, 3)
    return (r_point, t_point, x_out)
